# Initial kernel scaffold; baseline (speedup 1.0000x reference)
import jax
import jax.numpy as jnp
from jax.experimental import pallas as pl


def kernel(x, edge_attr, edge_index, W0, b0, We1, be1, We2, be2, conv_b, gru_Wih, gru_Whh, gru_bih, gru_bhh, lstm_Wih0, lstm_Whh0, lstm_bih0, lstm_bhh0, lstm_Wih1, lstm_Whh1, lstm_bih1, lstm_bhh1, lstm_Wih2, lstm_Whh2, lstm_bih2, lstm_bhh2, W1, b1, W2, b2):
    raise NotImplementedError("write your pallas kernel here")


# speedup vs baseline: 1.4006x; 1.4006x over previous
import functools

import jax
import jax.numpy as jnp
from jax import lax
from jax.experimental import pallas as pl
from jax.experimental.pallas import tpu as pltpu
from jax.experimental.pallas import tpu_sc as plsc

F32 = jnp.float32

N = 10000
E = 160000
DIN = 128
DE = 16
H = 32
EH = 128
OUT = 12
STEPS = 6
S2S_STEPS = 6
S2S_LAYERS = 3

NC = 2
NS = 16
NW = NC * NS
CH = 128
CPR = 4
ROUNDS = 10
W128 = 128
EPW = ROUNDS * CPR * CH
E_PAD = NW * EPW
N_PAD = 10240
HALF = N_PAD // 2
NPC = HALF + 128
DUMP = HALF
EPT = E_PAD // NS
SCHUNK = EPT // CH
SROUNDS = SCHUNK // CPR
ZRPT = NPC // NS
ORPT = HALF // NS

NCHUNK = EPW // CH


def _sc_gather_body(nodes_hbm, srcr_hbm, g_hbm, idx_v, buf_v, sem):
    cid = lax.axis_index("c")
    sid = lax.axis_index("s")
    wid = cid * NS + sid
    pltpu.sync_copy(srcr_hbm.at[pl.ds(wid * NCHUNK, NCHUNK)], idx_v)

    def round_body(r, _):
        descs = [
            pltpu.async_copy(nodes_hbm.at[idx_v.at[r * CPR + j]],
                             buf_v.at[pl.ds(j * CH, CH)], sem)
            for j in range(CPR)
        ]
        for d in descs:
            d.wait()
        pltpu.sync_copy(buf_v,
                        g_hbm.at[pl.ds(wid * EPW + r * CPR * CH, CPR * CH)])
        return _

    lax.fori_loop(0, ROUNDS, round_body, 0)


def _sc_scatter_body(msg_hbm, dstr_hbm, zeros_hbm, part_hbm, idx_v, buf_v,
                     acc_s):
    cid = lax.axis_index("c")
    sid = lax.axis_index("s")
    pltpu.sync_copy(dstr_hbm.at[cid, pl.ds(sid * SCHUNK, SCHUNK)], idx_v)
    pltpu.sync_copy(zeros_hbm.at[pl.ds(sid * ZRPT, ZRPT)],
                    acc_s.at[pl.ds(sid * ZRPT, ZRPT)])
    plsc.subcore_barrier()

    def round_body(r, _):
        pltpu.sync_copy(msg_hbm.at[pl.ds(sid * EPT + r * CPR * CH, CPR * CH)],
                        buf_v)
        for j in range(CPR):
            pltpu.sync_copy(buf_v.at[pl.ds(j * CH, CH)],
                            acc_s.at[idx_v.at[r * CPR + j]], add=True)
        return _

    lax.fori_loop(0, SROUNDS, round_body, 0)
    plsc.subcore_barrier()
    pltpu.sync_copy(acc_s.at[pl.ds(sid * ORPT, ORPT)],
                    part_hbm.at[cid, pl.ds(sid * ORPT, ORPT)])


@functools.lru_cache(maxsize=None)
def _sc_fns():
    mesh = plsc.VectorSubcoreMesh(core_axis_name="c", subcore_axis_name="s",
                                  num_cores=NC, num_subcores=NS)
    gather = pl.kernel(
        _sc_gather_body,
        out_type=jax.ShapeDtypeStruct((E_PAD, W128), F32),
        mesh=mesh,
        scratch_types=[
            pltpu.VMEM((NCHUNK, CH), jnp.int32),
            pltpu.VMEM((CPR * CH, W128), F32),
            pltpu.SemaphoreType.DMA,
        ],
    )
    scatter = pl.kernel(
        _sc_scatter_body,
        out_type=jax.ShapeDtypeStruct((NC, HALF, W128), F32),
        mesh=mesh,
        scratch_types=[
            pltpu.VMEM((SCHUNK, CH), jnp.int32),
            pltpu.VMEM((CPR * CH, W128), F32),
            pltpu.VMEM_SHARED((NPC, W128), F32),
        ],
    )
    return gather, scatter


def _prep_body(x_ref, w_ref, b_ref, o_ref):
    o_ref[...] = jnp.maximum(
        jnp.dot(x_ref[...], w_ref[...], preferred_element_type=F32)
        + b_ref[...], 0.0)


def _msg_body(ea_ref, g_ref, we1_ref, be1_ref, we2_ref, be2_ref, r_ref, s_ref,
              o_ref):
    a = jnp.maximum(
        jnp.dot(ea_ref[...], we1_ref[...], preferred_element_type=F32)
        + be1_ref[...], 0.0)
    y = jnp.dot(a, we2_ref[...], preferred_element_type=F32) + be2_ref[...]
    gb = jnp.dot(g_ref[...], r_ref[...], preferred_element_type=F32)
    o_ref[...] = jnp.dot(gb * y, s_ref[...], preferred_element_type=F32)


def _gru_body(p_ref, h_ref, cb_ref,
              wr_ref, wz_ref, wn_ref, ur_ref, uz_ref, un_ref,
              br_ref, bz_ref, bn_ref, hr_ref, hz_ref, hn_ref, o_ref):
    agg = jnp.concatenate(
        [p_ref[0, :, :H], p_ref[1, :N - HALF, :H]], axis=0)
    m = jnp.maximum(agg + cb_ref[...], 0.0)
    h = h_ref[:, :H]
    ir = jnp.dot(m, wr_ref[...], preferred_element_type=F32) + br_ref[...]
    iz = jnp.dot(m, wz_ref[...], preferred_element_type=F32) + bz_ref[...]
    inn = jnp.dot(m, wn_ref[...], preferred_element_type=F32) + bn_ref[...]
    hr = jnp.dot(h, ur_ref[...], preferred_element_type=F32) + hr_ref[...]
    hz = jnp.dot(h, uz_ref[...], preferred_element_type=F32) + hz_ref[...]
    hn = jnp.dot(h, un_ref[...], preferred_element_type=F32) + hn_ref[...]
    r = jax.nn.sigmoid(ir + hr)
    z = jax.nn.sigmoid(iz + hz)
    n = jnp.tanh(inn + r * hn)
    res = (1.0 - z) * n + z * h
    o_ref[...] = jnp.concatenate(
        [res, jnp.zeros((N, W128 - H), F32)], axis=1)


def _s2s_body(out_ref, *refs):
    y_ref = refs[-1]
    prm = list(refs[:-1])
    out = out_ref[:, :H]
    q_star = jnp.zeros((1, 2 * H), F32)
    hs = [jnp.zeros((1, H), F32) for _ in range(S2S_LAYERS)]
    cs = [jnp.zeros((1, H), F32) for _ in range(S2S_LAYERS)]
    for _ in range(S2S_STEPS):
        inp_l = q_star
        for l in range(S2S_LAYERS):
            p = prm[l * 12:(l + 1) * 12]
            i = jax.nn.sigmoid(
                jnp.dot(inp_l, p[0][...], preferred_element_type=F32)
                + jnp.dot(hs[l], p[4][...], preferred_element_type=F32)
                + p[8][...])
            f = jax.nn.sigmoid(
                jnp.dot(inp_l, p[1][...], preferred_element_type=F32)
                + jnp.dot(hs[l], p[5][...], preferred_element_type=F32)
                + p[9][...])
            g = jnp.tanh(
                jnp.dot(inp_l, p[2][...], preferred_element_type=F32)
                + jnp.dot(hs[l], p[6][...], preferred_element_type=F32)
                + p[10][...])
            o = jax.nn.sigmoid(
                jnp.dot(inp_l, p[3][...], preferred_element_type=F32)
                + jnp.dot(hs[l], p[7][...], preferred_element_type=F32)
                + p[11][...])
            cs[l] = f * cs[l] + i * g
            hs[l] = o * jnp.tanh(cs[l])
            inp_l = hs[l]
        q = hs[-1]
        e = jnp.sum(out * q, axis=1, keepdims=True)
        e = e - jnp.max(e)
        a = jnp.exp(e)
        alpha = a / jnp.sum(a)
        readout = jnp.sum(alpha * out, axis=0, keepdims=True)
        q_star = jnp.concatenate([q, readout], axis=1)
    w1, b1, w2, b2 = prm[-4:]
    y = jnp.maximum(
        jnp.dot(q_star, w1[...], preferred_element_type=F32) + b1[...], 0.0)
    y_ref[...] = jnp.dot(y, w2[...], preferred_element_type=F32) + b2[...]


def _row(v):
    return v.reshape(1, -1)


def kernel(x, edge_attr, edge_index, W0, b0, We1, be1, We2, be2, conv_b,
           gru_Wih, gru_Whh, gru_bih, gru_bhh,
           lstm_Wih0, lstm_Whh0, lstm_bih0, lstm_bhh0,
           lstm_Wih1, lstm_Whh1, lstm_bih1, lstm_bhh1,
           lstm_Wih2, lstm_Whh2, lstm_bih2, lstm_bhh2,
           W1, b1, W2, b2):
    src = edge_index[0].astype(jnp.int32)
    dst = edge_index[1].astype(jnp.int32)
    src_r = jnp.concatenate(
        [src, jnp.zeros((E_PAD - E,), jnp.int32)]).reshape(E_PAD // CH, CH)
    dst_pad = jnp.concatenate(
        [dst, jnp.full((E_PAD - E,), 2 * N_PAD, jnp.int32)])
    dst_r = jnp.stack([
        jnp.where((dst_pad >= c * HALF) & (dst_pad < (c + 1) * HALF),
                  dst_pad - c * HALF, DUMP).reshape(E_PAD // CH, CH)
        for c in range(NC)])
    ea_pad = jnp.concatenate(
        [edge_attr, jnp.zeros((E_PAD - E, DE), F32)], axis=0)
    zeros_np = jnp.zeros((NPC, W128), F32)

    eyeH = jnp.eye(H, dtype=F32)
    R_mat = jnp.repeat(eyeH, H, axis=1)
    S_mat = jnp.tile(eyeH, (H, 1))
    R128 = jnp.concatenate([R_mat, jnp.zeros((W128 - H, H * H), F32)], axis=0)
    S128 = jnp.concatenate([S_mat, jnp.zeros((H * H, W128 - H), F32)], axis=1)
    W0p = jnp.concatenate([W0, jnp.zeros((DIN, W128 - H), F32)], axis=1)
    b0p = jnp.concatenate([b0, jnp.zeros((W128 - H,), F32)])

    WihT = gru_Wih.T
    WhhT = gru_Whh.T
    wr, wz, wn = WihT[:, :H], WihT[:, H:2 * H], WihT[:, 2 * H:]
    ur, uz, un = WhhT[:, :H], WhhT[:, H:2 * H], WhhT[:, 2 * H:]
    br, bz, bn = (_row(gru_bih[:H]), _row(gru_bih[H:2 * H]),
                  _row(gru_bih[2 * H:]))
    hr, hz, hn = (_row(gru_bhh[:H]), _row(gru_bhh[H:2 * H]),
                  _row(gru_bhh[2 * H:]))

    lstm = [(lstm_Wih0, lstm_Whh0, lstm_bih0, lstm_bhh0),
            (lstm_Wih1, lstm_Whh1, lstm_bih1, lstm_bhh1),
            (lstm_Wih2, lstm_Whh2, lstm_bih2, lstm_bhh2)]
    s2s_prm = []
    for (Wih, Whh, bih, bhh) in lstm:
        WiT = Wih.T
        WhT = Whh.T
        b = _row(bih + bhh)
        s2s_prm += [WiT[:, k * H:(k + 1) * H] for k in range(4)]
        s2s_prm += [WhT[:, k * H:(k + 1) * H] for k in range(4)]
        s2s_prm += [b[:, k * H:(k + 1) * H] for k in range(4)]
    s2s_prm += [W1, _row(b1), W2, _row(b2)]

    out = pl.pallas_call(
        _prep_body,
        out_shape=jax.ShapeDtypeStruct((N, W128), F32),
    )(x, W0p, _row(b0p))
    h = out

    B = 1024
    grid = (E_PAD // B,)
    msg_call = pl.pallas_call(
        _msg_body,
        grid=grid,
        in_specs=[
            pl.BlockSpec((B, DE), lambda i: (i, 0)),
            pl.BlockSpec((B, W128), lambda i: (i, 0)),
            pl.BlockSpec((DE, EH), lambda i: (0, 0)),
            pl.BlockSpec((1, EH), lambda i: (0, 0)),
            pl.BlockSpec((EH, H * H), lambda i: (0, 0)),
            pl.BlockSpec((1, H * H), lambda i: (0, 0)),
            pl.BlockSpec((W128, H * H), lambda i: (0, 0)),
            pl.BlockSpec((H * H, W128), lambda i: (0, 0)),
        ],
        out_specs=pl.BlockSpec((B, W128), lambda i: (i, 0)),
        out_shape=jax.ShapeDtypeStruct((E_PAD, W128), F32),
    )
    gru_call = pl.pallas_call(
        _gru_body,
        out_shape=jax.ShapeDtypeStruct((N, W128), F32),
    )

    sc_gather, sc_scatter = _sc_fns()
    for _ in range(STEPS):
        g = sc_gather(out, src_r)
        msg = msg_call(ea_pad, g, We1, _row(be1), We2, _row(be2), R128, S128)
        parts = sc_scatter(msg, dst_r, zeros_np)
        h = gru_call(parts, h, _row(conv_b),
                     wr, wz, wn, ur, uz, un, br, bz, bn, hr, hz, hn)
        out = h

    y = pl.pallas_call(
        _s2s_body,
        out_shape=jax.ShapeDtypeStruct((1, OUT), F32),
    )(out, *s2s_prm)
    return y


# speedup vs baseline: 1.4202x; 1.0140x over previous
import functools

import jax
import jax.numpy as jnp
from jax import lax
from jax.experimental import pallas as pl
from jax.experimental.pallas import tpu as pltpu
from jax.experimental.pallas import tpu_sc as plsc

F32 = jnp.float32

N = 10000
E = 160000
DIN = 128
DE = 16
H = 32
EH = 128
OUT = 12
STEPS = 6
S2S_STEPS = 6
S2S_LAYERS = 3

NC = 2
NS = 16
NW = NC * NS
CH = 128
CPR = 4
ROUNDS = 10
W128 = 128
EPW = ROUNDS * CPR * CH
E_PAD = NW * EPW
N_PAD = 10240
HALF = N_PAD // 2
NPC = HALF + 128
DUMP = HALF
EPT = E_PAD // NS
SCHUNK = EPT // CH
SROUNDS = SCHUNK // CPR
ZRPT = NPC // NS
ORPT = HALF // NS

NCHUNK = EPW // CH


def _sc_gather_body(nodes_hbm, srcf_hbm, g_hbm, idx_v, buf_v, sem):
    cid = lax.axis_index("c")
    sid = lax.axis_index("s")
    wid = cid * NS + sid

    def round_body(r, _):
        base = wid * EPW + r * CPR * CH
        pltpu.sync_copy(srcf_hbm.at[pl.ds(base, CPR * CH)], idx_v)
        pltpu.async_copy(nodes_hbm.at[idx_v], buf_v, sem).wait()
        pltpu.sync_copy(buf_v, g_hbm.at[pl.ds(base, CPR * CH)])
        return _

    lax.fori_loop(0, ROUNDS, round_body, 0)


def _sc_scatter_body(msg_hbm, dstf_hbm, zeros_hbm, part_hbm, idx_v, buf_v,
                     acc_s):
    cid = lax.axis_index("c")
    sid = lax.axis_index("s")
    pltpu.sync_copy(zeros_hbm.at[pl.ds(sid * ZRPT, ZRPT)],
                    acc_s.at[pl.ds(sid * ZRPT, ZRPT)])
    plsc.subcore_barrier()

    def round_body(r, _):
        base = sid * EPT + r * CPR * CH
        pltpu.sync_copy(dstf_hbm.at[pl.ds(cid * E_PAD + base, CPR * CH)],
                        idx_v)
        pltpu.sync_copy(msg_hbm.at[pl.ds(base, CPR * CH)], buf_v)
        pltpu.sync_copy(buf_v, acc_s.at[idx_v], add=True)
        return _

    lax.fori_loop(0, SROUNDS, round_body, 0)
    plsc.subcore_barrier()
    pltpu.sync_copy(acc_s.at[pl.ds(sid * ORPT, ORPT)],
                    part_hbm.at[cid, pl.ds(sid * ORPT, ORPT)])


@functools.lru_cache(maxsize=None)
def _sc_fns():
    mesh = plsc.VectorSubcoreMesh(core_axis_name="c", subcore_axis_name="s",
                                  num_cores=NC, num_subcores=NS)
    gather = pl.kernel(
        _sc_gather_body,
        out_type=jax.ShapeDtypeStruct((E_PAD, W128), F32),
        mesh=mesh,
        scratch_types=[
            pltpu.VMEM((CPR * CH,), jnp.int32),
            pltpu.VMEM((CPR * CH, W128), F32),
            pltpu.SemaphoreType.DMA,
        ],
    )
    scatter = pl.kernel(
        _sc_scatter_body,
        out_type=jax.ShapeDtypeStruct((NC, HALF, W128), F32),
        mesh=mesh,
        scratch_types=[
            pltpu.VMEM((CPR * CH,), jnp.int32),
            pltpu.VMEM((CPR * CH, W128), F32),
            pltpu.VMEM_SHARED((NPC, W128), F32),
        ],
    )
    return gather, scatter


def _prep_body(x_ref, w_ref, b_ref, o_ref):
    o_ref[...] = jnp.maximum(
        jnp.dot(x_ref[...], w_ref[...], preferred_element_type=F32)
        + b_ref[...], 0.0)


def _msg_body(ea_ref, g_ref, we1_ref, be1_ref, we2_ref, be2_ref, r_ref, s_ref,
              o_ref):
    a = jnp.maximum(
        jnp.dot(ea_ref[...], we1_ref[...], preferred_element_type=F32)
        + be1_ref[...], 0.0)
    y = jnp.dot(a, we2_ref[...], preferred_element_type=F32) + be2_ref[...]
    gb = jnp.dot(g_ref[...], r_ref[...], preferred_element_type=F32)
    o_ref[...] = jnp.dot(gb * y, s_ref[...], preferred_element_type=F32)


def _gru_body(p_ref, h_ref, cb_ref,
              wr_ref, wz_ref, wn_ref, ur_ref, uz_ref, un_ref,
              br_ref, bz_ref, bn_ref, hr_ref, hz_ref, hn_ref, o_ref):
    agg = jnp.concatenate(
        [p_ref[0, :, :H], p_ref[1, :N - HALF, :H]], axis=0)
    m = jnp.maximum(agg + cb_ref[...], 0.0)
    h = h_ref[:, :H]
    ir = jnp.dot(m, wr_ref[...], preferred_element_type=F32) + br_ref[...]
    iz = jnp.dot(m, wz_ref[...], preferred_element_type=F32) + bz_ref[...]
    inn = jnp.dot(m, wn_ref[...], preferred_element_type=F32) + bn_ref[...]
    hr = jnp.dot(h, ur_ref[...], preferred_element_type=F32) + hr_ref[...]
    hz = jnp.dot(h, uz_ref[...], preferred_element_type=F32) + hz_ref[...]
    hn = jnp.dot(h, un_ref[...], preferred_element_type=F32) + hn_ref[...]
    r = jax.nn.sigmoid(ir + hr)
    z = jax.nn.sigmoid(iz + hz)
    n = jnp.tanh(inn + r * hn)
    res = (1.0 - z) * n + z * h
    o_ref[...] = jnp.concatenate(
        [res, jnp.zeros((N, W128 - H), F32)], axis=1)


def _s2s_body(out_ref, *refs):
    y_ref = refs[-1]
    prm = list(refs[:-1])
    out = out_ref[:, :H]
    q_star = jnp.zeros((1, 2 * H), F32)
    hs = [jnp.zeros((1, H), F32) for _ in range(S2S_LAYERS)]
    cs = [jnp.zeros((1, H), F32) for _ in range(S2S_LAYERS)]
    for _ in range(S2S_STEPS):
        inp_l = q_star
        for l in range(S2S_LAYERS):
            p = prm[l * 12:(l + 1) * 12]
            i = jax.nn.sigmoid(
                jnp.dot(inp_l, p[0][...], preferred_element_type=F32)
                + jnp.dot(hs[l], p[4][...], preferred_element_type=F32)
                + p[8][...])
            f = jax.nn.sigmoid(
                jnp.dot(inp_l, p[1][...], preferred_element_type=F32)
                + jnp.dot(hs[l], p[5][...], preferred_element_type=F32)
                + p[9][...])
            g = jnp.tanh(
                jnp.dot(inp_l, p[2][...], preferred_element_type=F32)
                + jnp.dot(hs[l], p[6][...], preferred_element_type=F32)
                + p[10][...])
            o = jax.nn.sigmoid(
                jnp.dot(inp_l, p[3][...], preferred_element_type=F32)
                + jnp.dot(hs[l], p[7][...], preferred_element_type=F32)
                + p[11][...])
            cs[l] = f * cs[l] + i * g
            hs[l] = o * jnp.tanh(cs[l])
            inp_l = hs[l]
        q = hs[-1]
        e = jnp.sum(out * q, axis=1, keepdims=True)
        e = e - jnp.max(e)
        a = jnp.exp(e)
        alpha = a / jnp.sum(a)
        readout = jnp.sum(alpha * out, axis=0, keepdims=True)
        q_star = jnp.concatenate([q, readout], axis=1)
    w1, b1, w2, b2 = prm[-4:]
    y = jnp.maximum(
        jnp.dot(q_star, w1[...], preferred_element_type=F32) + b1[...], 0.0)
    y_ref[...] = jnp.dot(y, w2[...], preferred_element_type=F32) + b2[...]


def _row(v):
    return v.reshape(1, -1)


def kernel(x, edge_attr, edge_index, W0, b0, We1, be1, We2, be2, conv_b,
           gru_Wih, gru_Whh, gru_bih, gru_bhh,
           lstm_Wih0, lstm_Whh0, lstm_bih0, lstm_bhh0,
           lstm_Wih1, lstm_Whh1, lstm_bih1, lstm_bhh1,
           lstm_Wih2, lstm_Whh2, lstm_bih2, lstm_bhh2,
           W1, b1, W2, b2):
    src = edge_index[0].astype(jnp.int32)
    dst = edge_index[1].astype(jnp.int32)
    src_r = jnp.concatenate([src, jnp.zeros((E_PAD - E,), jnp.int32)])
    dst_pad = jnp.concatenate(
        [dst, jnp.full((E_PAD - E,), 2 * N_PAD, jnp.int32)])
    dst_r = jnp.concatenate([
        jnp.where((dst_pad >= c * HALF) & (dst_pad < (c + 1) * HALF),
                  dst_pad - c * HALF, DUMP)
        for c in range(NC)])
    ea_pad = jnp.concatenate(
        [edge_attr, jnp.zeros((E_PAD - E, DE), F32)], axis=0)
    zeros_np = jnp.zeros((NPC, W128), F32)

    eyeH = jnp.eye(H, dtype=F32)
    R_mat = jnp.repeat(eyeH, H, axis=1)
    S_mat = jnp.tile(eyeH, (H, 1))
    R128 = jnp.concatenate([R_mat, jnp.zeros((W128 - H, H * H), F32)], axis=0)
    S128 = jnp.concatenate([S_mat, jnp.zeros((H * H, W128 - H), F32)], axis=1)
    W0p = jnp.concatenate([W0, jnp.zeros((DIN, W128 - H), F32)], axis=1)
    b0p = jnp.concatenate([b0, jnp.zeros((W128 - H,), F32)])

    WihT = gru_Wih.T
    WhhT = gru_Whh.T
    wr, wz, wn = WihT[:, :H], WihT[:, H:2 * H], WihT[:, 2 * H:]
    ur, uz, un = WhhT[:, :H], WhhT[:, H:2 * H], WhhT[:, 2 * H:]
    br, bz, bn = (_row(gru_bih[:H]), _row(gru_bih[H:2 * H]),
                  _row(gru_bih[2 * H:]))
    hr, hz, hn = (_row(gru_bhh[:H]), _row(gru_bhh[H:2 * H]),
                  _row(gru_bhh[2 * H:]))

    lstm = [(lstm_Wih0, lstm_Whh0, lstm_bih0, lstm_bhh0),
            (lstm_Wih1, lstm_Whh1, lstm_bih1, lstm_bhh1),
            (lstm_Wih2, lstm_Whh2, lstm_bih2, lstm_bhh2)]
    s2s_prm = []
    for (Wih, Whh, bih, bhh) in lstm:
        WiT = Wih.T
        WhT = Whh.T
        b = _row(bih + bhh)
        s2s_prm += [WiT[:, k * H:(k + 1) * H] for k in range(4)]
        s2s_prm += [WhT[:, k * H:(k + 1) * H] for k in range(4)]
        s2s_prm += [b[:, k * H:(k + 1) * H] for k in range(4)]
    s2s_prm += [W1, _row(b1), W2, _row(b2)]

    out = pl.pallas_call(
        _prep_body,
        out_shape=jax.ShapeDtypeStruct((N, W128), F32),
    )(x, W0p, _row(b0p))
    h = out

    B = 1024
    grid = (E_PAD // B,)
    msg_call = pl.pallas_call(
        _msg_body,
        grid=grid,
        in_specs=[
            pl.BlockSpec((B, DE), lambda i: (i, 0)),
            pl.BlockSpec((B, W128), lambda i: (i, 0)),
            pl.BlockSpec((DE, EH), lambda i: (0, 0)),
            pl.BlockSpec((1, EH), lambda i: (0, 0)),
            pl.BlockSpec((EH, H * H), lambda i: (0, 0)),
            pl.BlockSpec((1, H * H), lambda i: (0, 0)),
            pl.BlockSpec((W128, H * H), lambda i: (0, 0)),
            pl.BlockSpec((H * H, W128), lambda i: (0, 0)),
        ],
        out_specs=pl.BlockSpec((B, W128), lambda i: (i, 0)),
        out_shape=jax.ShapeDtypeStruct((E_PAD, W128), F32),
    )
    gru_call = pl.pallas_call(
        _gru_body,
        out_shape=jax.ShapeDtypeStruct((N, W128), F32),
    )

    sc_gather, sc_scatter = _sc_fns()
    for _ in range(STEPS):
        g = sc_gather(out, src_r)
        msg = msg_call(ea_pad, g, We1, _row(be1), We2, _row(be2), R128, S128)
        parts = sc_scatter(msg, dst_r, zeros_np)
        h = gru_call(parts, h, _row(conv_b),
                     wr, wz, wn, ur, uz, un, br, bz, bn, hr, hz, hn)
        out = h

    y = pl.pallas_call(
        _s2s_body,
        out_shape=jax.ShapeDtypeStruct((1, OUT), F32),
    )(out, *s2s_prm)
    return y


# speedup vs baseline: 1.4543x; 1.0240x over previous
import functools

import jax
import jax.numpy as jnp
from jax import lax
from jax.experimental import pallas as pl
from jax.experimental.pallas import tpu as pltpu
from jax.experimental.pallas import tpu_sc as plsc

F32 = jnp.float32

N = 10000
E = 160000
DIN = 128
DE = 16
H = 32
EH = 128
OUT = 12
STEPS = 6
S2S_STEPS = 6
S2S_LAYERS = 3

NC = 2
NS = 16
NW = NC * NS
CH = 128
CPR = 4
ROUNDS = 10
W128 = 128
EPW = ROUNDS * CPR * CH
E_PAD = NW * EPW
N_PAD = 10240
HALF = N_PAD // 2
NPC = HALF + 128
DUMP = HALF
EPT = E_PAD // NS
SCHUNK = EPT // CH
SROUNDS = SCHUNK // CPR
ZRPT = NPC // NS
ORPT = HALF // NS

NCHUNK = EPW // CH


GROWS = 640
GROUNDS = EPW // GROWS
SROWS = 512


def _sc_gather_body(nodes_hbm, srcf_hbm, g_hbm, idx_v, buf_v, sem):
    cid = lax.axis_index("c")
    sid = lax.axis_index("s")
    wid = cid * NS + sid
    base = wid * EPW
    pltpu.sync_copy(srcf_hbm.at[pl.ds(base, EPW)], idx_v)

    def round_body(r, _):
        pltpu.async_copy(nodes_hbm.at[idx_v.at[pl.ds(r * GROWS, GROWS)]],
                         buf_v, sem).wait()
        pltpu.sync_copy(buf_v, g_hbm.at[pl.ds(base + r * GROWS, GROWS)])
        return _

    lax.fori_loop(0, GROUNDS, round_body, 0)


def _sc_scatter_body(msg_hbm, dstf_hbm, zeros_hbm, part_hbm, idx_v, buf_v,
                     acc_s):
    cid = lax.axis_index("c")
    sid = lax.axis_index("s")
    base = sid * EPT
    pltpu.sync_copy(zeros_hbm.at[pl.ds(sid * ZRPT, ZRPT)],
                    acc_s.at[pl.ds(sid * ZRPT, ZRPT)])
    plsc.subcore_barrier()

    def round_body(r, _):
        pltpu.sync_copy(dstf_hbm.at[pl.ds(cid * E_PAD + base + r * SROWS,
                                          SROWS)], idx_v)
        pltpu.sync_copy(msg_hbm.at[pl.ds(base + r * SROWS, SROWS)], buf_v)
        pltpu.sync_copy(buf_v, acc_s.at[idx_v], add=True)
        return _

    lax.fori_loop(0, EPT // SROWS, round_body, 0)
    plsc.subcore_barrier()
    pltpu.sync_copy(acc_s.at[pl.ds(sid * ORPT, ORPT)],
                    part_hbm.at[cid, pl.ds(sid * ORPT, ORPT)])


@functools.lru_cache(maxsize=None)
def _sc_fns():
    mesh = plsc.VectorSubcoreMesh(core_axis_name="c", subcore_axis_name="s",
                                  num_cores=NC, num_subcores=NS)
    gather = pl.kernel(
        _sc_gather_body,
        out_type=jax.ShapeDtypeStruct((E_PAD, W128), F32),
        mesh=mesh,
        scratch_types=[
            pltpu.VMEM((EPW,), jnp.int32),
            pltpu.VMEM((GROWS, W128), F32),
            pltpu.SemaphoreType.DMA,
        ],
    )
    scatter = pl.kernel(
        _sc_scatter_body,
        out_type=jax.ShapeDtypeStruct((NC, HALF, W128), F32),
        mesh=mesh,
        scratch_types=[
            pltpu.VMEM((SROWS,), jnp.int32),
            pltpu.VMEM((SROWS, W128), F32),
            pltpu.VMEM_SHARED((NPC, W128), F32),
        ],
    )
    return gather, scatter


def _prep_body(x_ref, w_ref, b_ref, o_ref):
    o_ref[...] = jnp.maximum(
        jnp.dot(x_ref[...], w_ref[...], preferred_element_type=F32)
        + b_ref[...], 0.0)


BF16 = jnp.bfloat16


def _msg_body(ea_ref, g_ref, we1_ref, be1_ref, we2_ref, be2_ref, r_ref, s_ref,
              o_ref):
    a = jnp.maximum(
        jnp.dot(ea_ref[...], we1_ref[...], preferred_element_type=F32)
        + be1_ref[...], 0.0)
    y = jnp.dot(a, we2_ref[...], preferred_element_type=F32) + be2_ref[...]
    gb = jnp.dot(g_ref[:, :H], r_ref[...], preferred_element_type=F32)
    msg = jnp.dot(gb * y, s_ref[...], preferred_element_type=F32)
    o_ref[...] = jnp.concatenate(
        [msg, jnp.zeros((msg.shape[0], W128 - H), F32)], axis=1)


def _gru_body(p_ref, h_ref, cb_ref,
              wr_ref, wz_ref, wn_ref, ur_ref, uz_ref, un_ref,
              br_ref, bz_ref, bn_ref, hr_ref, hz_ref, hn_ref, o_ref):
    agg = jnp.concatenate(
        [p_ref[0, :, :H], p_ref[1, :N - HALF, :H]], axis=0)
    m = jnp.maximum(agg + cb_ref[...], 0.0)
    h = h_ref[:, :H]
    ir = jnp.dot(m, wr_ref[...], preferred_element_type=F32) + br_ref[...]
    iz = jnp.dot(m, wz_ref[...], preferred_element_type=F32) + bz_ref[...]
    inn = jnp.dot(m, wn_ref[...], preferred_element_type=F32) + bn_ref[...]
    hr = jnp.dot(h, ur_ref[...], preferred_element_type=F32) + hr_ref[...]
    hz = jnp.dot(h, uz_ref[...], preferred_element_type=F32) + hz_ref[...]
    hn = jnp.dot(h, un_ref[...], preferred_element_type=F32) + hn_ref[...]
    r = jax.nn.sigmoid(ir + hr)
    z = jax.nn.sigmoid(iz + hz)
    n = jnp.tanh(inn + r * hn)
    res = (1.0 - z) * n + z * h
    o_ref[...] = jnp.concatenate(
        [res, jnp.zeros((N, W128 - H), F32)], axis=1)


def _s2s_body(out_ref, *refs):
    y_ref = refs[-1]
    prm = list(refs[:-1])
    out = out_ref[:, :H]
    q_star = jnp.zeros((1, 2 * H), F32)
    hs = [jnp.zeros((1, H), F32) for _ in range(S2S_LAYERS)]
    cs = [jnp.zeros((1, H), F32) for _ in range(S2S_LAYERS)]
    for _ in range(S2S_STEPS):
        inp_l = q_star
        for l in range(S2S_LAYERS):
            p = prm[l * 12:(l + 1) * 12]
            i = jax.nn.sigmoid(
                jnp.dot(inp_l, p[0][...], preferred_element_type=F32)
                + jnp.dot(hs[l], p[4][...], preferred_element_type=F32)
                + p[8][...])
            f = jax.nn.sigmoid(
                jnp.dot(inp_l, p[1][...], preferred_element_type=F32)
                + jnp.dot(hs[l], p[5][...], preferred_element_type=F32)
                + p[9][...])
            g = jnp.tanh(
                jnp.dot(inp_l, p[2][...], preferred_element_type=F32)
                + jnp.dot(hs[l], p[6][...], preferred_element_type=F32)
                + p[10][...])
            o = jax.nn.sigmoid(
                jnp.dot(inp_l, p[3][...], preferred_element_type=F32)
                + jnp.dot(hs[l], p[7][...], preferred_element_type=F32)
                + p[11][...])
            cs[l] = f * cs[l] + i * g
            hs[l] = o * jnp.tanh(cs[l])
            inp_l = hs[l]
        q = hs[-1]
        e = jnp.sum(out * q, axis=1, keepdims=True)
        e = e - jnp.max(e)
        a = jnp.exp(e)
        alpha = a / jnp.sum(a)
        readout = jnp.sum(alpha * out, axis=0, keepdims=True)
        q_star = jnp.concatenate([q, readout], axis=1)
    w1, b1, w2, b2 = prm[-4:]
    y = jnp.maximum(
        jnp.dot(q_star, w1[...], preferred_element_type=F32) + b1[...], 0.0)
    y_ref[...] = jnp.dot(y, w2[...], preferred_element_type=F32) + b2[...]


def _row(v):
    return v.reshape(1, -1)


def kernel(x, edge_attr, edge_index, W0, b0, We1, be1, We2, be2, conv_b,
           gru_Wih, gru_Whh, gru_bih, gru_bhh,
           lstm_Wih0, lstm_Whh0, lstm_bih0, lstm_bhh0,
           lstm_Wih1, lstm_Whh1, lstm_bih1, lstm_bhh1,
           lstm_Wih2, lstm_Whh2, lstm_bih2, lstm_bhh2,
           W1, b1, W2, b2):
    src = edge_index[0].astype(jnp.int32)
    dst = edge_index[1].astype(jnp.int32)
    src_r = jnp.concatenate([src, jnp.zeros((E_PAD - E,), jnp.int32)])
    dst_pad = jnp.concatenate(
        [dst, jnp.full((E_PAD - E,), 2 * N_PAD, jnp.int32)])
    dst_r = jnp.concatenate([
        jnp.where((dst_pad >= c * HALF) & (dst_pad < (c + 1) * HALF),
                  dst_pad - c * HALF, DUMP)
        for c in range(NC)])
    ea_pad = jnp.concatenate(
        [edge_attr, jnp.zeros((E_PAD - E, DE), F32)], axis=0)
    zeros_np = jnp.zeros((NPC, W128), F32)

    eyeH = jnp.eye(H, dtype=F32)
    R_mat = jnp.repeat(eyeH, H, axis=1)
    S_mat = jnp.tile(eyeH, (H, 1))
    W0p = jnp.concatenate([W0, jnp.zeros((DIN, W128 - H), F32)], axis=1)
    b0p = jnp.concatenate([b0, jnp.zeros((W128 - H,), F32)])

    WihT = gru_Wih.T
    WhhT = gru_Whh.T
    wr, wz, wn = WihT[:, :H], WihT[:, H:2 * H], WihT[:, 2 * H:]
    ur, uz, un = WhhT[:, :H], WhhT[:, H:2 * H], WhhT[:, 2 * H:]
    br, bz, bn = (_row(gru_bih[:H]), _row(gru_bih[H:2 * H]),
                  _row(gru_bih[2 * H:]))
    hr, hz, hn = (_row(gru_bhh[:H]), _row(gru_bhh[H:2 * H]),
                  _row(gru_bhh[2 * H:]))

    lstm = [(lstm_Wih0, lstm_Whh0, lstm_bih0, lstm_bhh0),
            (lstm_Wih1, lstm_Whh1, lstm_bih1, lstm_bhh1),
            (lstm_Wih2, lstm_Whh2, lstm_bih2, lstm_bhh2)]
    s2s_prm = []
    for (Wih, Whh, bih, bhh) in lstm:
        WiT = Wih.T
        WhT = Whh.T
        b = _row(bih + bhh)
        s2s_prm += [WiT[:, k * H:(k + 1) * H] for k in range(4)]
        s2s_prm += [WhT[:, k * H:(k + 1) * H] for k in range(4)]
        s2s_prm += [b[:, k * H:(k + 1) * H] for k in range(4)]
    s2s_prm += [W1, _row(b1), W2, _row(b2)]

    out = pl.pallas_call(
        _prep_body,
        out_shape=jax.ShapeDtypeStruct((N, W128), F32),
    )(x, W0p, _row(b0p))
    h = out

    B = 1024
    grid = (E_PAD // B,)
    msg_call = pl.pallas_call(
        _msg_body,
        grid=grid,
        in_specs=[
            pl.BlockSpec((B, DE), lambda i: (i, 0)),
            pl.BlockSpec((B, W128), lambda i: (i, 0)),
            pl.BlockSpec((DE, EH), lambda i: (0, 0)),
            pl.BlockSpec((1, EH), lambda i: (0, 0)),
            pl.BlockSpec((EH, H * H), lambda i: (0, 0)),
            pl.BlockSpec((1, H * H), lambda i: (0, 0)),
            pl.BlockSpec((H, H * H), lambda i: (0, 0)),
            pl.BlockSpec((H * H, H), lambda i: (0, 0)),
        ],
        out_specs=pl.BlockSpec((B, W128), lambda i: (i, 0)),
        out_shape=jax.ShapeDtypeStruct((E_PAD, W128), F32),
    )
    gru_call = pl.pallas_call(
        _gru_body,
        out_shape=jax.ShapeDtypeStruct((N, W128), F32),
    )

    sc_gather, sc_scatter = _sc_fns()
    for _ in range(STEPS):
        g = sc_gather(out, src_r)
        msg = msg_call(ea_pad, g, We1, _row(be1), We2, _row(be2),
                       R_mat, S_mat)
        parts = sc_scatter(msg, dst_r, zeros_np)
        h = gru_call(parts, h, _row(conv_b),
                     wr, wz, wn, ur, uz, un, br, bz, bn, hr, hz, hn)
        out = h

    y = pl.pallas_call(
        _s2s_body,
        out_shape=jax.ShapeDtypeStruct((1, OUT), F32),
    )(out, *s2s_prm)
    return y


# speedup vs baseline: 1.5131x; 1.0404x over previous
import functools

import jax
import jax.numpy as jnp
from jax import lax
from jax.experimental import pallas as pl
from jax.experimental.pallas import tpu as pltpu
from jax.experimental.pallas import tpu_sc as plsc

F32 = jnp.float32

N = 10000
E = 160000
DIN = 128
DE = 16
H = 32
EH = 128
OUT = 12
STEPS = 6
S2S_STEPS = 6
S2S_LAYERS = 3

NC = 2
NS = 16
NW = NC * NS
CH = 128
CPR = 4
ROUNDS = 10
W128 = 128
EPW = ROUNDS * CPR * CH
E_PAD = NW * EPW
N_PAD = 10240
HALF = N_PAD // 2
NPC = HALF + 128
DUMP = HALF
EPT = E_PAD // NS
SCHUNK = EPT // CH
SROUNDS = SCHUNK // CPR
ZRPT = NPC // NS
ORPT = HALF // NS

NCHUNK = EPW // CH


GROWS = 640
GROUNDS = EPW // GROWS
SROWS = 512


def _sc_gather_body(nodes_hbm, srcf_hbm, g_hbm, idx_v, buf_v, sem):
    cid = lax.axis_index("c")
    sid = lax.axis_index("s")
    wid = cid * NS + sid
    base = wid * EPW
    pltpu.sync_copy(srcf_hbm.at[pl.ds(base, EPW)], idx_v)

    def round_body(r, _):
        pltpu.async_copy(nodes_hbm.at[idx_v.at[pl.ds(r * GROWS, GROWS)]],
                         buf_v, sem).wait()
        pltpu.sync_copy(buf_v, g_hbm.at[pl.ds(base + r * GROWS, GROWS)])
        return _

    lax.fori_loop(0, GROUNDS, round_body, 0)


SROWS2 = 256
NRS = EPT // SROWS2


def _sc_scatter_body(msg_hbm, dstf_hbm, zeros_hbm, part_hbm,
                     idx0, idx1, buf0, buf1, acc_s, sa0, sa1, si0, si1):
    cid = lax.axis_index("c")
    sid = lax.axis_index("s")
    base = sid * EPT
    ibase = cid * E_PAD + base

    def slab(r):
        return msg_hbm.at[pl.ds(base + r * SROWS2, SROWS2)]

    def islab(r):
        return dstf_hbm.at[pl.ds(ibase + r * SROWS2, SROWS2)]

    pltpu.sync_copy(zeros_hbm.at[pl.ds(sid * ZRPT, ZRPT)],
                    acc_s.at[pl.ds(sid * ZRPT, ZRPT)])
    plsc.subcore_barrier()

    pltpu.async_copy(slab(0), buf0, sa0)
    pltpu.async_copy(islab(0), idx0, si0)

    def round_body(k, _):
        for b, (ib, bb, sa, si, ibn, bbn, san, sin) in enumerate([
                (idx0, buf0, sa0, si0, idx1, buf1, sa1, si1),
                (idx1, buf1, sa1, si1, idx0, buf0, sa0, si0)]):
            r = 2 * k + b
            rn = jnp.minimum(r + 1, NRS - 1)
            pltpu.async_copy(slab(rn), bbn, san)
            pltpu.async_copy(islab(rn), ibn, sin)
            pltpu.make_async_copy(slab(r), bb, sa).wait()
            pltpu.make_async_copy(islab(r), ib, si).wait()
            pltpu.sync_copy(bb, acc_s.at[ib], add=True)
        return _

    lax.fori_loop(0, NRS // 2, round_body, 0)
    pltpu.make_async_copy(slab(NRS - 1), buf0, sa0).wait()
    pltpu.make_async_copy(islab(NRS - 1), idx0, si0).wait()
    plsc.subcore_barrier()
    pltpu.sync_copy(acc_s.at[pl.ds(sid * ORPT, ORPT)],
                    part_hbm.at[cid, pl.ds(sid * ORPT, ORPT)])


@functools.lru_cache(maxsize=None)
def _sc_fns():
    mesh = plsc.VectorSubcoreMesh(core_axis_name="c", subcore_axis_name="s",
                                  num_cores=NC, num_subcores=NS)
    gather = pl.kernel(
        _sc_gather_body,
        out_type=jax.ShapeDtypeStruct((E_PAD, W128), F32),
        mesh=mesh,
        scratch_types=[
            pltpu.VMEM((EPW,), jnp.int32),
            pltpu.VMEM((GROWS, W128), F32),
            pltpu.SemaphoreType.DMA,
        ],
    )
    scatter = pl.kernel(
        _sc_scatter_body,
        out_type=jax.ShapeDtypeStruct((NC, HALF, W128), F32),
        mesh=mesh,
        scratch_types=[
            pltpu.VMEM((SROWS2,), jnp.int32),
            pltpu.VMEM((SROWS2,), jnp.int32),
            pltpu.VMEM((SROWS2, W128), F32),
            pltpu.VMEM((SROWS2, W128), F32),
            pltpu.VMEM_SHARED((NPC, W128), F32),
            pltpu.SemaphoreType.DMA,
            pltpu.SemaphoreType.DMA,
            pltpu.SemaphoreType.DMA,
            pltpu.SemaphoreType.DMA,
        ],
    )
    return gather, scatter


def _prep_body(x_ref, w_ref, b_ref, o_ref):
    o_ref[...] = jnp.maximum(
        jnp.dot(x_ref[...], w_ref[...], preferred_element_type=F32)
        + b_ref[...], 0.0)


BF16 = jnp.bfloat16


def _msg_body(ea_ref, g_ref, we1_ref, be1_ref, we2_ref, be2_ref, r_ref, s_ref,
              o_ref):
    a = jnp.maximum(
        jnp.dot(ea_ref[...], we1_ref[...], preferred_element_type=F32)
        + be1_ref[...], 0.0)
    y = jnp.dot(a, we2_ref[...], preferred_element_type=F32) + be2_ref[...]
    gb = jnp.dot(g_ref[:, :H], r_ref[...], preferred_element_type=F32)
    msg = jnp.dot(gb * y, s_ref[...], preferred_element_type=F32)
    o_ref[...] = jnp.concatenate(
        [msg, jnp.zeros((msg.shape[0], W128 - H), F32)], axis=1)


def _gru_body(p_ref, h_ref, cb_ref,
              wr_ref, wz_ref, wn_ref, ur_ref, uz_ref, un_ref,
              br_ref, bz_ref, bn_ref, hr_ref, hz_ref, hn_ref, o_ref):
    agg = jnp.concatenate(
        [p_ref[0, :, :H], p_ref[1, :N - HALF, :H]], axis=0)
    m = jnp.maximum(agg + cb_ref[...], 0.0)
    h = h_ref[:, :H]
    ir = jnp.dot(m, wr_ref[...], preferred_element_type=F32) + br_ref[...]
    iz = jnp.dot(m, wz_ref[...], preferred_element_type=F32) + bz_ref[...]
    inn = jnp.dot(m, wn_ref[...], preferred_element_type=F32) + bn_ref[...]
    hr = jnp.dot(h, ur_ref[...], preferred_element_type=F32) + hr_ref[...]
    hz = jnp.dot(h, uz_ref[...], preferred_element_type=F32) + hz_ref[...]
    hn = jnp.dot(h, un_ref[...], preferred_element_type=F32) + hn_ref[...]
    r = jax.nn.sigmoid(ir + hr)
    z = jax.nn.sigmoid(iz + hz)
    n = jnp.tanh(inn + r * hn)
    res = (1.0 - z) * n + z * h
    o_ref[...] = jnp.concatenate(
        [res, jnp.zeros((N, W128 - H), F32)], axis=1)


def _s2s_body(out_ref, *refs):
    y_ref = refs[-1]
    prm = list(refs[:-1])
    out = out_ref[:, :H]
    q_star = jnp.zeros((1, 2 * H), F32)
    hs = [jnp.zeros((1, H), F32) for _ in range(S2S_LAYERS)]
    cs = [jnp.zeros((1, H), F32) for _ in range(S2S_LAYERS)]
    for _ in range(S2S_STEPS):
        inp_l = q_star
        for l in range(S2S_LAYERS):
            p = prm[l * 12:(l + 1) * 12]
            i = jax.nn.sigmoid(
                jnp.dot(inp_l, p[0][...], preferred_element_type=F32)
                + jnp.dot(hs[l], p[4][...], preferred_element_type=F32)
                + p[8][...])
            f = jax.nn.sigmoid(
                jnp.dot(inp_l, p[1][...], preferred_element_type=F32)
                + jnp.dot(hs[l], p[5][...], preferred_element_type=F32)
                + p[9][...])
            g = jnp.tanh(
                jnp.dot(inp_l, p[2][...], preferred_element_type=F32)
                + jnp.dot(hs[l], p[6][...], preferred_element_type=F32)
                + p[10][...])
            o = jax.nn.sigmoid(
                jnp.dot(inp_l, p[3][...], preferred_element_type=F32)
                + jnp.dot(hs[l], p[7][...], preferred_element_type=F32)
                + p[11][...])
            cs[l] = f * cs[l] + i * g
            hs[l] = o * jnp.tanh(cs[l])
            inp_l = hs[l]
        q = hs[-1]
        e = jnp.sum(out * q, axis=1, keepdims=True)
        e = e - jnp.max(e)
        a = jnp.exp(e)
        alpha = a / jnp.sum(a)
        readout = jnp.sum(alpha * out, axis=0, keepdims=True)
        q_star = jnp.concatenate([q, readout], axis=1)
    w1, b1, w2, b2 = prm[-4:]
    y = jnp.maximum(
        jnp.dot(q_star, w1[...], preferred_element_type=F32) + b1[...], 0.0)
    y_ref[...] = jnp.dot(y, w2[...], preferred_element_type=F32) + b2[...]


def _row(v):
    return v.reshape(1, -1)


def kernel(x, edge_attr, edge_index, W0, b0, We1, be1, We2, be2, conv_b,
           gru_Wih, gru_Whh, gru_bih, gru_bhh,
           lstm_Wih0, lstm_Whh0, lstm_bih0, lstm_bhh0,
           lstm_Wih1, lstm_Whh1, lstm_bih1, lstm_bhh1,
           lstm_Wih2, lstm_Whh2, lstm_bih2, lstm_bhh2,
           W1, b1, W2, b2):
    src = edge_index[0].astype(jnp.int32)
    dst = edge_index[1].astype(jnp.int32)
    src_r = jnp.concatenate([src, jnp.zeros((E_PAD - E,), jnp.int32)])
    dst_pad = jnp.concatenate(
        [dst, jnp.full((E_PAD - E,), 2 * N_PAD, jnp.int32)])
    dst_r = jnp.concatenate([
        jnp.where((dst_pad >= c * HALF) & (dst_pad < (c + 1) * HALF),
                  dst_pad - c * HALF, DUMP)
        for c in range(NC)])
    ea_pad = jnp.concatenate(
        [edge_attr, jnp.zeros((E_PAD - E, DE), F32)], axis=0)
    zeros_np = jnp.zeros((NPC, W128), F32)

    eyeH = jnp.eye(H, dtype=F32)
    R_mat = jnp.repeat(eyeH, H, axis=1)
    S_mat = jnp.tile(eyeH, (H, 1))
    W0p = jnp.concatenate([W0, jnp.zeros((DIN, W128 - H), F32)], axis=1)
    b0p = jnp.concatenate([b0, jnp.zeros((W128 - H,), F32)])

    WihT = gru_Wih.T
    WhhT = gru_Whh.T
    wr, wz, wn = WihT[:, :H], WihT[:, H:2 * H], WihT[:, 2 * H:]
    ur, uz, un = WhhT[:, :H], WhhT[:, H:2 * H], WhhT[:, 2 * H:]
    br, bz, bn = (_row(gru_bih[:H]), _row(gru_bih[H:2 * H]),
                  _row(gru_bih[2 * H:]))
    hr, hz, hn = (_row(gru_bhh[:H]), _row(gru_bhh[H:2 * H]),
                  _row(gru_bhh[2 * H:]))

    lstm = [(lstm_Wih0, lstm_Whh0, lstm_bih0, lstm_bhh0),
            (lstm_Wih1, lstm_Whh1, lstm_bih1, lstm_bhh1),
            (lstm_Wih2, lstm_Whh2, lstm_bih2, lstm_bhh2)]
    s2s_prm = []
    for (Wih, Whh, bih, bhh) in lstm:
        WiT = Wih.T
        WhT = Whh.T
        b = _row(bih + bhh)
        s2s_prm += [WiT[:, k * H:(k + 1) * H] for k in range(4)]
        s2s_prm += [WhT[:, k * H:(k + 1) * H] for k in range(4)]
        s2s_prm += [b[:, k * H:(k + 1) * H] for k in range(4)]
    s2s_prm += [W1, _row(b1), W2, _row(b2)]

    out = pl.pallas_call(
        _prep_body,
        out_shape=jax.ShapeDtypeStruct((N, W128), F32),
    )(x, W0p, _row(b0p))
    h = out

    B = 1024
    grid = (E_PAD // B,)
    msg_call = pl.pallas_call(
        _msg_body,
        grid=grid,
        in_specs=[
            pl.BlockSpec((B, DE), lambda i: (i, 0)),
            pl.BlockSpec((B, W128), lambda i: (i, 0)),
            pl.BlockSpec((DE, EH), lambda i: (0, 0)),
            pl.BlockSpec((1, EH), lambda i: (0, 0)),
            pl.BlockSpec((EH, H * H), lambda i: (0, 0)),
            pl.BlockSpec((1, H * H), lambda i: (0, 0)),
            pl.BlockSpec((H, H * H), lambda i: (0, 0)),
            pl.BlockSpec((H * H, H), lambda i: (0, 0)),
        ],
        out_specs=pl.BlockSpec((B, W128), lambda i: (i, 0)),
        out_shape=jax.ShapeDtypeStruct((E_PAD, W128), F32),
    )
    gru_call = pl.pallas_call(
        _gru_body,
        out_shape=jax.ShapeDtypeStruct((N, W128), F32),
    )

    sc_gather, sc_scatter = _sc_fns()
    for _ in range(STEPS):
        g = sc_gather(out, src_r)
        msg = msg_call(ea_pad, g, We1, _row(be1), We2, _row(be2),
                       R_mat, S_mat)
        parts = sc_scatter(msg, dst_r, zeros_np)
        h = gru_call(parts, h, _row(conv_b),
                     wr, wz, wn, ur, uz, un, br, bz, bn, hr, hz, hn)
        out = h

    y = pl.pallas_call(
        _s2s_body,
        out_shape=jax.ShapeDtypeStruct((1, OUT), F32),
    )(out, *s2s_prm)
    return y
